# baseline (device time: 18071 ns/iter reference)
import jax
import jax.numpy as jnp
from jax import lax
from jax.experimental import pallas as pl
from jax.experimental.pallas import tpu as pltpu


def kernel(x, pi):
    m, h, w = x.shape

    def body(pi_ref, x_ref, out_ref, send_sem, recv_sem):
        my_x = lax.axis_index("x")
        my_y = lax.axis_index("y")
        dst = pi_ref[my_x]

        barrier_sem = pltpu.get_barrier_semaphore()
        pl.semaphore_signal(
            barrier_sem,
            inc=1,
            device_id=(1 - my_x, my_y),
            device_id_type=pl.DeviceIdType.MESH,
        )
        pl.semaphore_wait(barrier_sem, 1)

        @pl.when(dst == my_x)
        def _():
            out_ref[...] = x_ref[...]

        @pl.when(dst != my_x)
        def _():
            rdma = pltpu.make_async_remote_copy(
                src_ref=x_ref,
                dst_ref=out_ref,
                send_sem=send_sem,
                recv_sem=recv_sem,
                device_id=(dst, my_y),
                device_id_type=pl.DeviceIdType.MESH,
            )
            rdma.start()
            rdma.wait()

    return pl.pallas_call(
        body,
        out_shape=jax.ShapeDtypeStruct((m, h, w), jnp.float32),
        in_specs=[
            pl.BlockSpec(memory_space=pltpu.SMEM),
            pl.BlockSpec(memory_space=pltpu.VMEM),
        ],
        out_specs=pl.BlockSpec(memory_space=pltpu.VMEM),
        scratch_shapes=[
            pltpu.SemaphoreType.DMA,
            pltpu.SemaphoreType.DMA,
        ],
        compiler_params=pltpu.CompilerParams(collective_id=0),
    )(pi, x)


# device time: 16377 ns/iter; 1.1034x vs baseline; 1.1034x over previous
import jax
import jax.numpy as jnp
from jax import lax
from jax.experimental import pallas as pl
from jax.experimental.pallas import tpu as pltpu

_CHUNKS = 4


def kernel(x, pi):
    m, h, w = x.shape
    half = h // 2
    rows = half // _CHUNKS

    def body(pi_ref, x_ref, out_ref, x_send, x_recv, y_send, y_recv):
        my_x = lax.axis_index("x")
        my_y = lax.axis_index("y")
        dst = pi_ref[my_x]

        barrier_sem = pltpu.get_barrier_semaphore()
        for nbr in [(1 - my_x, my_y), (my_x, 1 - my_y)]:
            pl.semaphore_signal(
                barrier_sem,
                inc=1,
                device_id=nbr,
                device_id_type=pl.DeviceIdType.MESH,
            )
        pl.semaphore_wait(barrier_sem, 2)

        @pl.when(dst == my_x)
        def _():
            out_ref[...] = x_ref[...]

        @pl.when(dst != my_x)
        def _():
            my_half0 = my_y * half

            def chunk(ref, k):
                return ref.at[:, pl.ds(my_half0 + k * rows, rows), :]

            x_rdmas = []
            for k in range(_CHUNKS):
                r = pltpu.make_async_remote_copy(
                    src_ref=chunk(x_ref, k),
                    dst_ref=chunk(out_ref, k),
                    send_sem=x_send.at[k],
                    recv_sem=x_recv.at[k],
                    device_id=(dst, my_y),
                    device_id_type=pl.DeviceIdType.MESH,
                )
                r.start()
                x_rdmas.append(r)

            y_rdmas = []
            for k in range(_CHUNKS):
                x_rdmas[k].wait_recv()
                r = pltpu.make_async_remote_copy(
                    src_ref=chunk(out_ref, k),
                    dst_ref=chunk(out_ref, k),
                    send_sem=y_send.at[k],
                    recv_sem=y_recv.at[k],
                    device_id=(my_x, 1 - my_y),
                    device_id_type=pl.DeviceIdType.MESH,
                )
                r.start()
                y_rdmas.append(r)

            for k in range(_CHUNKS):
                x_rdmas[k].wait_send()
                y_rdmas[k].wait()

    return pl.pallas_call(
        body,
        out_shape=jax.ShapeDtypeStruct((m, h, w), jnp.float32),
        in_specs=[
            pl.BlockSpec(memory_space=pltpu.SMEM),
            pl.BlockSpec(memory_space=pltpu.VMEM),
        ],
        out_specs=pl.BlockSpec(memory_space=pltpu.VMEM),
        scratch_shapes=[
            pltpu.SemaphoreType.DMA((_CHUNKS,)),
            pltpu.SemaphoreType.DMA((_CHUNKS,)),
            pltpu.SemaphoreType.DMA((_CHUNKS,)),
            pltpu.SemaphoreType.DMA((_CHUNKS,)),
        ],
        compiler_params=pltpu.CompilerParams(collective_id=0),
    )(pi, x)
